# baseline (speedup 1.0000x reference)
import jax
import jax.numpy as jnp
from jax import lax
from jax.experimental import pallas as pl
from jax.experimental.pallas import tpu as pltpu
from jax.experimental.pallas import tpu_sc as plsc

H = 1024; I = 4096; MI = 512; E = 8; K = 2; EPS = 1e-06; B = 1; S = 2048
T = B * S
TT = 256
IT = 512
NC, NS, L = 2, 16, 16
NW = NC * NS
CHA = T * K // NW
CHT = T // NW
CAP = T
TILE_G = 512
NB = CAP // TILE_G
_NT = (((1,), (1,)), ((), ()))


def _sc_mesh():
    return plsc.VectorSubcoreMesh(core_axis_name="c", subcore_axis_name="s",
                                  num_cores=NC, num_subcores=NS)


def _stage_a_body(x_ref, plw_ref, pl2w_ref, rwt_ref, rs_ref, pes_ref,
                  h1_ref, h2f_ref, ti_ref, tw_ref):
    x = x_ref[...]
    ms = jnp.mean(x * x, axis=1, keepdims=True) + EPS
    xn = x * lax.rsqrt(ms)
    h1_ref[...] = (xn * plw_ref[...]).astype(jnp.bfloat16)
    h2f_ref[...] = xn * pl2w_ref[...]
    xr = xn * rs_ref[...] * (H ** -0.5)
    logits = jnp.dot(xr, rwt_ref[...], preferred_element_type=jnp.float32,
                     precision=lax.Precision.DEFAULT)
    m = jnp.max(logits, axis=1, keepdims=True)
    p = jnp.exp(logits - m)
    probs = p / jnp.sum(p, axis=1, keepdims=True)
    iota = lax.broadcasted_iota(jnp.int32, probs.shape, 1)
    m1 = jnp.max(probs, axis=1, keepdims=True)
    i1 = jnp.min(jnp.where(probs == m1, iota, E), axis=1, keepdims=True)
    probs2 = jnp.where(iota == i1, -jnp.inf, probs)
    m2 = jnp.max(probs2, axis=1, keepdims=True)
    i2 = jnp.min(jnp.where(probs2 == m2, iota, E), axis=1, keepdims=True)
    s = m1 + m2
    pes = pes_ref[...]
    pes1 = jnp.sum(jnp.where(iota == i1, pes, 0.0), axis=1, keepdims=True)
    pes2 = jnp.sum(jnp.where(iota == i2, pes, 0.0), axis=1, keepdims=True)
    ti_ref[...] = jnp.concatenate([i1, i2], axis=1)
    tw_ref[...] = jnp.concatenate([m1 / s * pes1, m2 / s * pes2], axis=1)


def _glu_body(h1_ref, gt_ref, ut_ref, dt_ref, out_ref):
    i = pl.program_id(0)
    h1 = h1_ref[...]
    gb = gt_ref[...].astype(jnp.bfloat16)
    ub = ut_ref[...].astype(jnp.bfloat16)
    db = dt_ref[...].astype(jnp.bfloat16)
    g = lax.dot_general(h1, gb, _NT, preferred_element_type=jnp.float32)
    u = lax.dot_general(h1, ub, _NT, preferred_element_type=jnp.float32)
    hm = (jax.nn.gelu(g, approximate=True) * u).astype(jnp.bfloat16)
    contrib = lax.dot_general(hm, db, _NT, preferred_element_type=jnp.float32)

    @pl.when(i == 0)
    def _():
        out_ref[...] = contrib

    @pl.when(i > 0)
    def _():
        out_ref[...] += contrib


def _sc_wid():
    return lax.axis_index("s") * NC + lax.axis_index("c")


def _d1_hist_body(tif_ref, hist_ref, ti_v, histv_v):
    wid = _sc_wid()
    pltpu.sync_copy(tif_ref.at[pl.ds(wid * CHA, CHA)], ti_v)
    lane = lax.broadcasted_iota(jnp.int32, (L,), 0)
    histv = jnp.zeros((L,), jnp.int32)
    for e in range(E):
        tot = jnp.int32(0)
        for v in range(CHA // L):
            eid = ti_v[pl.ds(v * L, L)]
            tot = tot + jnp.sum(jnp.where(eid == e, 1, 0))
        histv = jnp.where(lane == e, tot, histv)
    histv_v[...] = histv
    pltpu.sync_copy(histv_v, hist_ref.at[wid])


def _d2_dispatch_body(tif_ref, h2f_ref, hist_ref, xd_ref, slots_ref,
                      counts_ref, ti_v, slots_v, hist_v, rows_v, sk0_v, sk1_v,
                      cnt_v, sem):
    wid = _sc_wid()
    pltpu.sync_copy(hist_ref, hist_v)
    pltpu.sync_copy(tif_ref.at[pl.ds(wid * CHA, CHA)], ti_v)
    wiota = lax.broadcasted_iota(jnp.int32, (L,), 0)
    lane = wiota
    bases = []
    countsv = jnp.zeros((L,), jnp.int32)
    for e in range(E):
        esplat = jnp.full((L,), e, jnp.int32)
        lo = plsc.load_gather(hist_v, [wiota, esplat])
        hi = plsc.load_gather(hist_v, [wiota + L, esplat])
        pref = (jnp.sum(jnp.where(wiota < wid, lo, 0)) +
                jnp.sum(jnp.where(wiota + L < wid, hi, 0)))
        bases.append(e * CAP + pref)
        countsv = jnp.where(lane == e, jnp.sum(lo) + jnp.sum(hi), countsv)

    run = [jnp.int32(0) for _ in range(E)]
    ones = jnp.ones((L,), jnp.int32)
    for v in range(CHA // L):
        eid = ti_v[pl.ds(v * L, L)]
        slotv = jnp.zeros((L,), jnp.int32)
        for e in range(E):
            mask = eid == e
            ind = jnp.where(mask, ones, 0)
            rank = plsc.cumsum(ind) - 1
            slotv = jnp.where(mask, bases[e] + run[e] + rank, slotv)
            run[e] = run[e] + jnp.sum(ind)
        slots_v[pl.ds(v * L, L)] = slotv

    pltpu.sync_copy(slots_v, slots_ref.at[pl.ds(wid * CHA, CHA)])

    @pl.when(wid == 0)
    def _():
        cnt_v[...] = countsv
        pltpu.sync_copy(cnt_v, counts_ref)

    evn = wiota * 2
    for j in range(CHT // L):
        sk0_v[pl.ds(j * L, L)] = plsc.load_gather(slots_v, [evn + 2 * L * j])
        sk1_v[pl.ds(j * L, L)] = plsc.load_gather(slots_v, [evn + 2 * L * j + 1])

    pltpu.sync_copy(h2f_ref.at[pl.ds(wid * CHT, CHT)], rows_v)
    c0 = pltpu.async_copy(rows_v, xd_ref.at[sk0_v], sem)
    c1 = pltpu.async_copy(rows_v, xd_ref.at[sk1_v], sem)
    c0.wait()
    c1.wait()


def _grouped_body(cnt_ref, xd_ref, egt_ref, eut_ref, edt_ref, out_ref):
    j = pl.program_id(1)

    @pl.when(j * TILE_G < cnt_ref[pl.program_id(0)])
    def _():
        xb = xd_ref[...].astype(jnp.bfloat16)
        eg = egt_ref[0].astype(jnp.bfloat16)
        eu = eut_ref[0].astype(jnp.bfloat16)
        ed = edt_ref[0].astype(jnp.bfloat16)
        g = lax.dot_general(xb, eg, _NT, preferred_element_type=jnp.float32)
        u = lax.dot_general(xb, eu, _NT, preferred_element_type=jnp.float32)
        hm = (jax.nn.gelu(g, approximate=True) * u).astype(jnp.bfloat16)
        out_ref[...] = lax.dot_general(hm, ed, _NT,
                                       preferred_element_type=jnp.float32)


def _d3_combine_body(slots_ref, twf_ref, od_ref, moe_ref, slots_v, tw_v,
                     sk0_v, sk1_v, rows0_v, rows1_v, out_v, sem):
    wid = _sc_wid()
    pltpu.sync_copy(slots_ref.at[pl.ds(wid * CHA, CHA)], slots_v)
    pltpu.sync_copy(twf_ref.at[pl.ds(wid * CHA, CHA)], tw_v)
    wiota = lax.broadcasted_iota(jnp.int32, (L,), 0)
    evn = wiota * 2
    HT = CHT // 2
    for half in range(2):
        for j in range(HT // L):
            off = half * CHA // 2 + 2 * L * j
            sk0_v[pl.ds(j * L, L)] = plsc.load_gather(slots_v, [evn + off])
            sk1_v[pl.ds(j * L, L)] = plsc.load_gather(slots_v, [evn + off + 1])
        c0 = pltpu.async_copy(od_ref.at[sk0_v], rows0_v, sem)
        c1 = pltpu.async_copy(od_ref.at[sk1_v], rows1_v, sem)
        c0.wait()
        c1.wait()
        UR = 4

        def row_body(rg, _):
            for u in range(UR):
                r = rg * UR + u
                a = half * CHA // 2 + 2 * r
                w0 = plsc.load_gather(tw_v, [jnp.full((L,), a, jnp.int32)])
                w1 = plsc.load_gather(tw_v, [jnp.full((L,), a + 1, jnp.int32)])
                for jj in range(H // L):
                    sl = pl.ds(jj * L, L)
                    out_v[r, sl] = w0 * rows0_v[r, sl] + w1 * rows1_v[r, sl]
            return 0

        lax.fori_loop(0, HT // UR, row_body, 0)
        pltpu.sync_copy(out_v, moe_ref.at[pl.ds(wid * CHT + half * HT, HT)])


def _final_body(mlp1_ref, mlp2_ref, moe_ref, p1_ref, p2_ref, pf_ref, out_ref):
    def rms(v, w):
        ms = jnp.mean(v * v, axis=1, keepdims=True) + EPS
        return v * lax.rsqrt(ms) * w

    a = rms(mlp1_ref[...] + mlp2_ref[...], p1_ref[...])
    b = rms(moe_ref[...], p2_ref[...])
    out_ref[...] = rms(a + b, pf_ref[...])


def kernel(hidden_states, pre_ln_w, pre_ln2_w, post_ln1_w, post_ln2_w,
           post_ln_w, gate_w, up_w, down_w, router_w, router_scale,
           per_expert_scale, exp_gate_w, exp_up_w, exp_down_w):
    bf16 = jnp.bfloat16
    f32 = jnp.float32
    x2d = hidden_states.reshape(T, H)

    n_tt = T // TT
    h1, h2f, ti, tw = pl.pallas_call(
        _stage_a_body,
        grid=(n_tt,),
        in_specs=[
            pl.BlockSpec((TT, H), lambda i: (i, 0)),
            pl.BlockSpec((1, H), lambda i: (0, 0)),
            pl.BlockSpec((1, H), lambda i: (0, 0)),
            pl.BlockSpec((H, E), lambda i: (0, 0)),
            pl.BlockSpec((1, H), lambda i: (0, 0)),
            pl.BlockSpec((1, E), lambda i: (0, 0)),
        ],
        out_specs=[
            pl.BlockSpec((TT, H), lambda i: (i, 0)),
            pl.BlockSpec((TT, H), lambda i: (i, 0)),
            pl.BlockSpec((TT, K), lambda i: (i, 0)),
            pl.BlockSpec((TT, K), lambda i: (i, 0)),
        ],
        out_shape=[
            jax.ShapeDtypeStruct((T, H), bf16),
            jax.ShapeDtypeStruct((T, H), f32),
            jax.ShapeDtypeStruct((T, K), jnp.int32),
            jax.ShapeDtypeStruct((T, K), f32),
        ],
    )(x2d, pre_ln_w.reshape(1, H), pre_ln2_w.reshape(1, H), router_w.T,
      router_scale.reshape(1, H), per_expert_scale.reshape(1, E))

    def glu_half(base):
        return pl.pallas_call(
            _glu_body,
            grid=(I // IT // 2,),
            in_specs=[
                pl.BlockSpec((T, H), lambda i: (0, 0)),
                pl.BlockSpec((IT, H), lambda i, b=base: (i + b, 0)),
                pl.BlockSpec((IT, H), lambda i, b=base: (i + b, 0)),
                pl.BlockSpec((H, IT), lambda i, b=base: (0, i + b)),
            ],
            out_specs=pl.BlockSpec((T, H), lambda i: (0, 0)),
            out_shape=jax.ShapeDtypeStruct((T, H), f32),
        )(h1, gate_w, up_w, down_w)

    tif = ti.reshape(T * K)
    twf = tw.reshape(T * K)

    hist = pl.kernel(
        _d1_hist_body,
        out_type=jax.ShapeDtypeStruct((NW, L), jnp.int32),
        mesh=_sc_mesh(),
        compiler_params=pltpu.CompilerParams(needs_layout_passes=False),
        scratch_types=[
            pltpu.VMEM((CHA,), jnp.int32),
            pltpu.VMEM((L,), jnp.int32),
        ],
    )(tif)

    mlp1 = glu_half(0)

    xd, slots, counts = pl.kernel(
        _d2_dispatch_body,
        out_type=[
            jax.ShapeDtypeStruct((E * CAP, H), f32),
            jax.ShapeDtypeStruct((T * K,), jnp.int32),
            jax.ShapeDtypeStruct((L,), jnp.int32),
        ],
        mesh=_sc_mesh(),
        compiler_params=pltpu.CompilerParams(needs_layout_passes=False),
        scratch_types=[
            pltpu.VMEM((CHA,), jnp.int32),
            pltpu.VMEM((CHA,), jnp.int32),
            pltpu.VMEM((NW, L), jnp.int32),
            pltpu.VMEM((CHT, H), f32),
            pltpu.VMEM((CHT,), jnp.int32),
            pltpu.VMEM((CHT,), jnp.int32),
            pltpu.VMEM((L,), jnp.int32),
            pltpu.SemaphoreType.DMA,
        ],
    )(tif, h2f, hist)

    grid_spec = pltpu.PrefetchScalarGridSpec(
        num_scalar_prefetch=1,
        grid=(E, NB),
        in_specs=[
            pl.BlockSpec(
                (TILE_G, H),
                lambda e, j, c: (
                    e * NB + jnp.minimum(
                        j, jnp.maximum((c[e] + TILE_G - 1) // TILE_G - 1, 0)),
                    0)),
            pl.BlockSpec((1, MI, H), lambda e, j, c: (e, 0, 0)),
            pl.BlockSpec((1, MI, H), lambda e, j, c: (e, 0, 0)),
            pl.BlockSpec((1, H, MI), lambda e, j, c: (e, 0, 0)),
        ],
        out_specs=pl.BlockSpec(
            (TILE_G, H),
            lambda e, j, c: (
                e * NB + jnp.minimum(
                    j, jnp.maximum((c[e] + TILE_G - 1) // TILE_G - 1, 0)),
                0)),
    )
    od = pl.pallas_call(
        _grouped_body,
        grid_spec=grid_spec,
        out_shape=jax.ShapeDtypeStruct((E * CAP, H), f32),
    )(counts, xd, exp_gate_w, exp_up_w, exp_down_w)

    mlp2 = glu_half(I // IT // 2)

    moe_raw = pl.kernel(
        _d3_combine_body,
        out_type=jax.ShapeDtypeStruct((T, H), f32),
        mesh=_sc_mesh(),
        compiler_params=pltpu.CompilerParams(needs_layout_passes=False),
        scratch_types=[
            pltpu.VMEM((CHA,), jnp.int32),
            pltpu.VMEM((CHA,), f32),
            pltpu.VMEM((CHT // 2,), jnp.int32),
            pltpu.VMEM((CHT // 2,), jnp.int32),
            pltpu.VMEM((CHT // 2, H), f32),
            pltpu.VMEM((CHT // 2, H), f32),
            pltpu.VMEM((CHT // 2, H), f32),
            pltpu.SemaphoreType.DMA,
        ],
    )(slots, twf, od)

    out = pl.pallas_call(
        _final_body,
        grid=(n_tt,),
        in_specs=[
            pl.BlockSpec((TT, H), lambda i: (i, 0)),
            pl.BlockSpec((TT, H), lambda i: (i, 0)),
            pl.BlockSpec((TT, H), lambda i: (i, 0)),
            pl.BlockSpec((1, H), lambda i: (0, 0)),
            pl.BlockSpec((1, H), lambda i: (0, 0)),
            pl.BlockSpec((1, H), lambda i: (0, 0)),
        ],
        out_specs=pl.BlockSpec((TT, H), lambda i: (i, 0)),
        out_shape=jax.ShapeDtypeStruct((T, H), f32),
    )(mlp1, mlp2, moe_raw, post_ln1_w.reshape(1, H), post_ln2_w.reshape(1, H),
      post_ln_w.reshape(1, H))

    return out.reshape(B, S, H)


# speedup vs baseline: 1.0780x; 1.0780x over previous
import jax
import jax.numpy as jnp
from jax import lax
from jax.experimental import pallas as pl
from jax.experimental.pallas import tpu as pltpu
from jax.experimental.pallas import tpu_sc as plsc

H = 1024; I = 4096; MI = 512; E = 8; K = 2; EPS = 1e-06; B = 1; S = 2048
T = B * S
TT = 256
IT = 512
NC, NS, L = 2, 16, 16
NW = NC * NS
CHA = T * K // NW
CHT = T // NW
CAP = T
TILE_G = 512
NB = CAP // TILE_G
_NT = (((1,), (1,)), ((), ()))


def _sc_mesh():
    return plsc.VectorSubcoreMesh(core_axis_name="c", subcore_axis_name="s",
                                  num_cores=NC, num_subcores=NS)


def _stage_a_body(x_ref, plw_ref, pl2w_ref, rwt_ref, rs_ref, pes_ref,
                  h1_ref, h2f_ref, ti_ref, tw_ref, hist_ref):
    x = x_ref[...]
    ms = jnp.mean(x * x, axis=1, keepdims=True) + EPS
    xn = x * lax.rsqrt(ms)
    h1_ref[...] = (xn * plw_ref[...]).astype(jnp.bfloat16)
    h2f_ref[...] = xn * pl2w_ref[...]
    xr = xn * rs_ref[...] * (H ** -0.5)
    logits = jnp.dot(xr, rwt_ref[...], preferred_element_type=jnp.float32,
                     precision=lax.Precision.DEFAULT)
    m = jnp.max(logits, axis=1, keepdims=True)
    p = jnp.exp(logits - m)
    probs = p / jnp.sum(p, axis=1, keepdims=True)
    iota = lax.broadcasted_iota(jnp.int32, probs.shape, 1)
    m1 = jnp.max(probs, axis=1, keepdims=True)
    i1 = jnp.min(jnp.where(probs == m1, iota, E), axis=1, keepdims=True)
    probs2 = jnp.where(iota == i1, -jnp.inf, probs)
    m2 = jnp.max(probs2, axis=1, keepdims=True)
    i2 = jnp.min(jnp.where(probs2 == m2, iota, E), axis=1, keepdims=True)
    s = m1 + m2
    pes = pes_ref[...]
    pes1 = jnp.sum(jnp.where(iota == i1, pes, 0.0), axis=1, keepdims=True)
    pes2 = jnp.sum(jnp.where(iota == i2, pes, 0.0), axis=1, keepdims=True)
    ti_ref[...] = jnp.concatenate([i1, i2], axis=1)
    tw_ref[...] = jnp.concatenate([m1 / s * pes1, m2 / s * pes2], axis=1)
    lane16 = lax.broadcasted_iota(jnp.int32, (TT, L), 1)
    oh = (jnp.where(lane16 == i1, 1, 0) + jnp.where(lane16 == i2, 1, 0))
    hist_ref[...] = oh.reshape(TT // CHT, CHT, L).sum(axis=1).reshape(
        1, TT // CHT, L)


def _glu_body(h1_ref, gt_ref, ut_ref, dt_ref, out_ref):
    i = pl.program_id(0)
    h1 = h1_ref[...]
    gb = gt_ref[...].astype(jnp.bfloat16)
    ub = ut_ref[...].astype(jnp.bfloat16)
    db = dt_ref[...].astype(jnp.bfloat16)
    g = lax.dot_general(h1, gb, _NT, preferred_element_type=jnp.float32)
    u = lax.dot_general(h1, ub, _NT, preferred_element_type=jnp.float32)
    hm = (jax.nn.gelu(g, approximate=True) * u).astype(jnp.bfloat16)
    contrib = lax.dot_general(hm, db, _NT, preferred_element_type=jnp.float32)

    @pl.when(i == 0)
    def _():
        out_ref[...] = contrib

    @pl.when(i > 0)
    def _():
        out_ref[...] += contrib


def _sc_wid():
    return lax.axis_index("s") * NC + lax.axis_index("c")


def _d1_hist_body(tif_ref, hist_ref, ti_v, histv_v):
    wid = _sc_wid()
    pltpu.sync_copy(tif_ref.at[pl.ds(wid * CHA, CHA)], ti_v)
    lane = lax.broadcasted_iota(jnp.int32, (L,), 0)
    histv = jnp.zeros((L,), jnp.int32)
    for e in range(E):
        tot = jnp.int32(0)
        for v in range(CHA // L):
            eid = ti_v[pl.ds(v * L, L)]
            tot = tot + jnp.sum(jnp.where(eid == e, 1, 0))
        histv = jnp.where(lane == e, tot, histv)
    histv_v[...] = histv
    pltpu.sync_copy(histv_v, hist_ref.at[wid])


def _d2_dispatch_body(tif_ref, h2f_ref, hist_ref, xd_ref, slots_ref,
                      counts_ref, ti_v, slots_v, hist_v, rows_v, sk0_v, sk1_v,
                      cnt_v, sem):
    wid = _sc_wid()
    pltpu.sync_copy(hist_ref, hist_v)
    pltpu.sync_copy(tif_ref.at[pl.ds(wid * CHA, CHA)], ti_v)
    wiota = lax.broadcasted_iota(jnp.int32, (L,), 0)
    lane = wiota
    bases = []
    countsv = jnp.zeros((L,), jnp.int32)
    for e in range(E):
        esplat = jnp.full((L,), e, jnp.int32)
        lo = plsc.load_gather(hist_v, [wiota, esplat])
        hi = plsc.load_gather(hist_v, [wiota + L, esplat])
        pref = (jnp.sum(jnp.where(wiota < wid, lo, 0)) +
                jnp.sum(jnp.where(wiota + L < wid, hi, 0)))
        bases.append(e * CAP + pref)
        countsv = jnp.where(lane == e, jnp.sum(lo) + jnp.sum(hi), countsv)

    run = [jnp.int32(0) for _ in range(E)]
    ones = jnp.ones((L,), jnp.int32)
    for v in range(CHA // L):
        eid = ti_v[pl.ds(v * L, L)]
        slotv = jnp.zeros((L,), jnp.int32)
        for e in range(E):
            mask = eid == e
            ind = jnp.where(mask, ones, 0)
            rank = plsc.cumsum(ind) - 1
            slotv = jnp.where(mask, bases[e] + run[e] + rank, slotv)
            run[e] = run[e] + jnp.sum(ind)
        slots_v[pl.ds(v * L, L)] = slotv

    pltpu.sync_copy(slots_v, slots_ref.at[pl.ds(wid * CHA, CHA)])

    @pl.when(wid == 0)
    def _():
        cnt_v[...] = countsv
        pltpu.sync_copy(cnt_v, counts_ref)

    evn = wiota * 2
    for j in range(CHT // L):
        sk0_v[pl.ds(j * L, L)] = plsc.load_gather(slots_v, [evn + 2 * L * j])
        sk1_v[pl.ds(j * L, L)] = plsc.load_gather(slots_v, [evn + 2 * L * j + 1])

    pltpu.sync_copy(h2f_ref.at[pl.ds(wid * CHT, CHT)], rows_v)
    c0 = pltpu.async_copy(rows_v, xd_ref.at[sk0_v], sem)
    c1 = pltpu.async_copy(rows_v, xd_ref.at[sk1_v], sem)
    c0.wait()
    c1.wait()


def _grouped_body(cnt_ref, xd_ref, egt_ref, eut_ref, edt_ref, out_ref):
    j = pl.program_id(1)

    @pl.when(j * TILE_G < cnt_ref[pl.program_id(0)])
    def _():
        xb = xd_ref[...].astype(jnp.bfloat16)
        eg = egt_ref[0].astype(jnp.bfloat16)
        eu = eut_ref[0].astype(jnp.bfloat16)
        ed = edt_ref[0].astype(jnp.bfloat16)
        g = lax.dot_general(xb, eg, _NT, preferred_element_type=jnp.float32)
        u = lax.dot_general(xb, eu, _NT, preferred_element_type=jnp.float32)
        hm = (jax.nn.gelu(g, approximate=True) * u).astype(jnp.bfloat16)
        out_ref[...] = lax.dot_general(hm, ed, _NT,
                                       preferred_element_type=jnp.float32)


def _d3_combine_body(slots_ref, twf_ref, od_ref, moe_ref, slots_v, tw_v,
                     sk0_v, sk1_v, rows0_v, rows1_v, out_v, sem):
    wid = _sc_wid()
    pltpu.sync_copy(slots_ref.at[pl.ds(wid * CHA, CHA)], slots_v)
    pltpu.sync_copy(twf_ref.at[pl.ds(wid * CHA, CHA)], tw_v)
    wiota = lax.broadcasted_iota(jnp.int32, (L,), 0)
    evn = wiota * 2
    HT = CHT // 2
    for half in range(2):
        for j in range(HT // L):
            off = half * CHA // 2 + 2 * L * j
            sk0_v[pl.ds(j * L, L)] = plsc.load_gather(slots_v, [evn + off])
            sk1_v[pl.ds(j * L, L)] = plsc.load_gather(slots_v, [evn + off + 1])
        c0 = pltpu.async_copy(od_ref.at[sk0_v], rows0_v, sem)
        c1 = pltpu.async_copy(od_ref.at[sk1_v], rows1_v, sem)
        c0.wait()
        c1.wait()
        UR = 4

        def row_body(rg, _):
            for u in range(UR):
                r = rg * UR + u
                a = half * CHA // 2 + 2 * r
                w0 = plsc.load_gather(tw_v, [jnp.full((L,), a, jnp.int32)])
                w1 = plsc.load_gather(tw_v, [jnp.full((L,), a + 1, jnp.int32)])
                for jj in range(H // L):
                    sl = pl.ds(jj * L, L)
                    out_v[r, sl] = w0 * rows0_v[r, sl] + w1 * rows1_v[r, sl]
            return 0

        lax.fori_loop(0, HT // UR, row_body, 0)
        pltpu.sync_copy(out_v, moe_ref.at[pl.ds(wid * CHT + half * HT, HT)])


def _final_body(mlp_ref, moe_ref, p1_ref, p2_ref, pf_ref, out_ref):
    def rms(v, w):
        ms = jnp.mean(v * v, axis=1, keepdims=True) + EPS
        return v * lax.rsqrt(ms) * w

    a = rms(mlp_ref[...], p1_ref[...])
    b = rms(moe_ref[...], p2_ref[...])
    out_ref[...] = rms(a + b, pf_ref[...])


def kernel(hidden_states, pre_ln_w, pre_ln2_w, post_ln1_w, post_ln2_w,
           post_ln_w, gate_w, up_w, down_w, router_w, router_scale,
           per_expert_scale, exp_gate_w, exp_up_w, exp_down_w):
    bf16 = jnp.bfloat16
    f32 = jnp.float32
    x2d = hidden_states.reshape(T, H)

    n_tt = T // TT
    h1, h2f, ti, tw, hist = pl.pallas_call(
        _stage_a_body,
        grid=(n_tt,),
        in_specs=[
            pl.BlockSpec((TT, H), lambda i: (i, 0)),
            pl.BlockSpec((1, H), lambda i: (0, 0)),
            pl.BlockSpec((1, H), lambda i: (0, 0)),
            pl.BlockSpec((H, E), lambda i: (0, 0)),
            pl.BlockSpec((1, H), lambda i: (0, 0)),
            pl.BlockSpec((1, E), lambda i: (0, 0)),
        ],
        out_specs=[
            pl.BlockSpec((TT, H), lambda i: (i, 0)),
            pl.BlockSpec((TT, H), lambda i: (i, 0)),
            pl.BlockSpec((TT, K), lambda i: (i, 0)),
            pl.BlockSpec((TT, K), lambda i: (i, 0)),
            pl.BlockSpec((1, TT // CHT, L), lambda i: (i, 0, 0)),
        ],
        out_shape=[
            jax.ShapeDtypeStruct((T, H), bf16),
            jax.ShapeDtypeStruct((T, H), f32),
            jax.ShapeDtypeStruct((T, K), jnp.int32),
            jax.ShapeDtypeStruct((T, K), f32),
            jax.ShapeDtypeStruct((T // TT, TT // CHT, L), jnp.int32),
        ],
    )(x2d, pre_ln_w.reshape(1, H), pre_ln2_w.reshape(1, H), router_w.T,
      router_scale.reshape(1, H), per_expert_scale.reshape(1, E))

    mlp_raw = pl.pallas_call(
        _glu_body,
        grid=(I // IT,),
        in_specs=[
            pl.BlockSpec((T, H), lambda i: (0, 0)),
            pl.BlockSpec((IT, H), lambda i: (i, 0)),
            pl.BlockSpec((IT, H), lambda i: (i, 0)),
            pl.BlockSpec((H, IT), lambda i: (0, i)),
        ],
        out_specs=pl.BlockSpec((T, H), lambda i: (0, 0)),
        out_shape=jax.ShapeDtypeStruct((T, H), f32),
    )(h1, gate_w, up_w, down_w)

    tif = ti.reshape(T * K)
    hist = hist.reshape(NW, L)
    twf = tw.reshape(T * K)

    xd, slots, counts = pl.kernel(
        _d2_dispatch_body,
        out_type=[
            jax.ShapeDtypeStruct((E * CAP, H), f32),
            jax.ShapeDtypeStruct((T * K,), jnp.int32),
            jax.ShapeDtypeStruct((L,), jnp.int32),
        ],
        mesh=_sc_mesh(),
        compiler_params=pltpu.CompilerParams(needs_layout_passes=False),
        scratch_types=[
            pltpu.VMEM((CHA,), jnp.int32),
            pltpu.VMEM((CHA,), jnp.int32),
            pltpu.VMEM((NW, L), jnp.int32),
            pltpu.VMEM((CHT, H), f32),
            pltpu.VMEM((CHT,), jnp.int32),
            pltpu.VMEM((CHT,), jnp.int32),
            pltpu.VMEM((L,), jnp.int32),
            pltpu.SemaphoreType.DMA,
        ],
    )(tif, h2f, hist)

    grid_spec = pltpu.PrefetchScalarGridSpec(
        num_scalar_prefetch=1,
        grid=(E, NB),
        in_specs=[
            pl.BlockSpec(
                (TILE_G, H),
                lambda e, j, c: (
                    e * NB + jnp.minimum(
                        j, jnp.maximum((c[e] + TILE_G - 1) // TILE_G - 1, 0)),
                    0)),
            pl.BlockSpec((1, MI, H), lambda e, j, c: (e, 0, 0)),
            pl.BlockSpec((1, MI, H), lambda e, j, c: (e, 0, 0)),
            pl.BlockSpec((1, H, MI), lambda e, j, c: (e, 0, 0)),
        ],
        out_specs=pl.BlockSpec(
            (TILE_G, H),
            lambda e, j, c: (
                e * NB + jnp.minimum(
                    j, jnp.maximum((c[e] + TILE_G - 1) // TILE_G - 1, 0)),
                0)),
    )
    od = pl.pallas_call(
        _grouped_body,
        grid_spec=grid_spec,
        out_shape=jax.ShapeDtypeStruct((E * CAP, H), f32),
    )(counts, xd, exp_gate_w, exp_up_w, exp_down_w)

    moe_raw = pl.kernel(
        _d3_combine_body,
        out_type=jax.ShapeDtypeStruct((T, H), f32),
        mesh=_sc_mesh(),
        compiler_params=pltpu.CompilerParams(needs_layout_passes=False),
        scratch_types=[
            pltpu.VMEM((CHA,), jnp.int32),
            pltpu.VMEM((CHA,), f32),
            pltpu.VMEM((CHT // 2,), jnp.int32),
            pltpu.VMEM((CHT // 2,), jnp.int32),
            pltpu.VMEM((CHT // 2, H), f32),
            pltpu.VMEM((CHT // 2, H), f32),
            pltpu.VMEM((CHT // 2, H), f32),
            pltpu.SemaphoreType.DMA,
        ],
    )(slots, twf, od)

    out = pl.pallas_call(
        _final_body,
        grid=(n_tt,),
        in_specs=[
            pl.BlockSpec((TT, H), lambda i: (i, 0)),
            pl.BlockSpec((TT, H), lambda i: (i, 0)),
            pl.BlockSpec((1, H), lambda i: (0, 0)),
            pl.BlockSpec((1, H), lambda i: (0, 0)),
            pl.BlockSpec((1, H), lambda i: (0, 0)),
        ],
        out_specs=pl.BlockSpec((TT, H), lambda i: (i, 0)),
        out_shape=jax.ShapeDtypeStruct((T, H), f32),
    )(mlp_raw, moe_raw, post_ln1_w.reshape(1, H), post_ln2_w.reshape(1, H),
      post_ln_w.reshape(1, H))

    return out.reshape(B, S, H)


# speedup vs baseline: 1.0825x; 1.0042x over previous
import jax
import jax.numpy as jnp
from jax import lax
from jax.experimental import pallas as pl
from jax.experimental.pallas import tpu as pltpu
from jax.experimental.pallas import tpu_sc as plsc

H = 1024; I = 4096; MI = 512; E = 8; K = 2; EPS = 1e-06; B = 1; S = 2048
T = B * S
TT = 256
IT = 512
NC, NS, L = 2, 16, 16
NW = NC * NS
CHA = T * K // NW
CHT = T // NW
CAP = T
TILE_G = 512
NB = CAP // TILE_G
_NT = (((1,), (1,)), ((), ()))


def _sc_mesh():
    return plsc.VectorSubcoreMesh(core_axis_name="c", subcore_axis_name="s",
                                  num_cores=NC, num_subcores=NS)


def _stage_a_body(x_ref, plw_ref, pl2w_ref, rwt_ref, rs_ref, pes_ref,
                  h1_ref, h2f_ref, ti_ref, tw_ref, hist_ref):
    x = x_ref[...]
    ms = jnp.mean(x * x, axis=1, keepdims=True) + EPS
    xn = x * lax.rsqrt(ms)
    h1_ref[...] = (xn * plw_ref[...]).astype(jnp.bfloat16)
    h2f_ref[...] = xn * pl2w_ref[...]
    xr = xn * rs_ref[...] * (H ** -0.5)
    logits = jnp.dot(xr, rwt_ref[...], preferred_element_type=jnp.float32,
                     precision=lax.Precision.DEFAULT)
    m = jnp.max(logits, axis=1, keepdims=True)
    p = jnp.exp(logits - m)
    probs = p / jnp.sum(p, axis=1, keepdims=True)
    iota = lax.broadcasted_iota(jnp.int32, probs.shape, 1)
    m1 = jnp.max(probs, axis=1, keepdims=True)
    i1 = jnp.min(jnp.where(probs == m1, iota, E), axis=1, keepdims=True)
    probs2 = jnp.where(iota == i1, -jnp.inf, probs)
    m2 = jnp.max(probs2, axis=1, keepdims=True)
    i2 = jnp.min(jnp.where(probs2 == m2, iota, E), axis=1, keepdims=True)
    s = m1 + m2
    pes = pes_ref[...]
    pes1 = jnp.sum(jnp.where(iota == i1, pes, 0.0), axis=1, keepdims=True)
    pes2 = jnp.sum(jnp.where(iota == i2, pes, 0.0), axis=1, keepdims=True)
    ti_ref[...] = jnp.concatenate([i1, i2], axis=1)
    tw_ref[...] = jnp.concatenate([m1 / s * pes1, m2 / s * pes2], axis=1)
    lane16 = lax.broadcasted_iota(jnp.int32, (TT, L), 1)
    oh = (jnp.where(lane16 == i1, 1, 0) + jnp.where(lane16 == i2, 1, 0))
    hist_ref[...] = oh.reshape(TT // CHT, CHT, L).sum(axis=1).reshape(
        1, TT // CHT, L)


def _glu_body(h1_ref, gt_ref, ut_ref, dt_ref, out_ref):
    i = pl.program_id(0)
    h1 = h1_ref[...]
    gb = gt_ref[...].astype(jnp.bfloat16)
    ub = ut_ref[...].astype(jnp.bfloat16)
    db = dt_ref[...].astype(jnp.bfloat16)
    g = lax.dot_general(h1, gb, _NT, preferred_element_type=jnp.float32)
    u = lax.dot_general(h1, ub, _NT, preferred_element_type=jnp.float32)
    hm = (jax.nn.gelu(g, approximate=True) * u).astype(jnp.bfloat16)
    contrib = lax.dot_general(hm, db, _NT, preferred_element_type=jnp.float32)

    @pl.when(i == 0)
    def _():
        out_ref[...] = contrib

    @pl.when(i > 0)
    def _():
        out_ref[...] += contrib


def _sc_wid():
    return lax.axis_index("s") * NC + lax.axis_index("c")


def _d1_hist_body(tif_ref, hist_ref, ti_v, histv_v):
    wid = _sc_wid()
    pltpu.sync_copy(tif_ref.at[pl.ds(wid * CHA, CHA)], ti_v)
    lane = lax.broadcasted_iota(jnp.int32, (L,), 0)
    histv = jnp.zeros((L,), jnp.int32)
    for e in range(E):
        tot = jnp.int32(0)
        for v in range(CHA // L):
            eid = ti_v[pl.ds(v * L, L)]
            tot = tot + jnp.sum(jnp.where(eid == e, 1, 0))
        histv = jnp.where(lane == e, tot, histv)
    histv_v[...] = histv
    pltpu.sync_copy(histv_v, hist_ref.at[wid])


def _d2_dispatch_body(tif_ref, h2f_ref, hist_ref, xd_ref, slots_ref,
                      counts_ref, ti_v, slots_v, hist_v, rows_v, sk0_v, sk1_v,
                      cnt_v, sem):
    wid = _sc_wid()
    crows = pltpu.async_copy(h2f_ref.at[pl.ds(wid * CHT, CHT)], rows_v, sem)
    pltpu.sync_copy(hist_ref, hist_v)
    pltpu.sync_copy(tif_ref.at[pl.ds(wid * CHA, CHA)], ti_v)
    wiota = lax.broadcasted_iota(jnp.int32, (L,), 0)
    lane = wiota
    bases = []
    countsv = jnp.zeros((L,), jnp.int32)
    for e in range(E):
        esplat = jnp.full((L,), e, jnp.int32)
        lo = plsc.load_gather(hist_v, [wiota, esplat])
        hi = plsc.load_gather(hist_v, [wiota + L, esplat])
        pref = (jnp.sum(jnp.where(wiota < wid, lo, 0)) +
                jnp.sum(jnp.where(wiota + L < wid, hi, 0)))
        bases.append(e * CAP + pref)
        countsv = jnp.where(lane == e, jnp.sum(lo) + jnp.sum(hi), countsv)

    run = [jnp.int32(0) for _ in range(E)]
    ones = jnp.ones((L,), jnp.int32)
    for v in range(CHA // L):
        eid = ti_v[pl.ds(v * L, L)]
        slotv = jnp.zeros((L,), jnp.int32)
        for e in range(E):
            mask = eid == e
            ind = jnp.where(mask, ones, 0)
            rank = plsc.cumsum(ind) - 1
            slotv = jnp.where(mask, bases[e] + run[e] + rank, slotv)
            run[e] = run[e] + jnp.sum(ind)
        slots_v[pl.ds(v * L, L)] = slotv

    pltpu.sync_copy(slots_v, slots_ref.at[pl.ds(wid * CHA, CHA)])

    @pl.when(wid == 0)
    def _():
        cnt_v[...] = countsv
        pltpu.sync_copy(cnt_v, counts_ref)

    evn = wiota * 2
    for j in range(CHT // L):
        sk0_v[pl.ds(j * L, L)] = plsc.load_gather(slots_v, [evn + 2 * L * j])
        sk1_v[pl.ds(j * L, L)] = plsc.load_gather(slots_v, [evn + 2 * L * j + 1])

    crows.wait()
    c0 = pltpu.async_copy(rows_v, xd_ref.at[sk0_v], sem)
    c1 = pltpu.async_copy(rows_v, xd_ref.at[sk1_v], sem)
    c0.wait()
    c1.wait()


def _grouped_body(cnt_ref, xd_ref, egt_ref, eut_ref, edt_ref, out_ref):
    j = pl.program_id(1)

    @pl.when(j * TILE_G < cnt_ref[pl.program_id(0)])
    def _():
        xb = xd_ref[...].astype(jnp.bfloat16)
        eg = egt_ref[0].astype(jnp.bfloat16)
        eu = eut_ref[0].astype(jnp.bfloat16)
        ed = edt_ref[0].astype(jnp.bfloat16)
        g = lax.dot_general(xb, eg, _NT, preferred_element_type=jnp.float32)
        u = lax.dot_general(xb, eu, _NT, preferred_element_type=jnp.float32)
        hm = (jax.nn.gelu(g, approximate=True) * u).astype(jnp.bfloat16)
        out_ref[...] = lax.dot_general(hm, ed, _NT,
                                       preferred_element_type=jnp.float32)


def _d3_combine_body(slots_ref, twf_ref, od_ref, moe_ref, slots_v, tw_v,
                     sk0_v, sk1_v, r0a_v, r0b_v, r1a_v, r1b_v, out_v, sem):
    wid = _sc_wid()
    pltpu.sync_copy(slots_ref.at[pl.ds(wid * CHA, CHA)], slots_v)
    pltpu.sync_copy(twf_ref.at[pl.ds(wid * CHA, CHA)], tw_v)
    wiota = lax.broadcasted_iota(jnp.int32, (L,), 0)
    evn = wiota * 2
    for j in range(CHT // L):
        sk0_v[pl.ds(j * L, L)] = plsc.load_gather(slots_v, [evn + 2 * L * j])
        sk1_v[pl.ds(j * L, L)] = plsc.load_gather(slots_v, [evn + 2 * L * j + 1])

    QT = L
    NQ = CHT // QT
    r0 = [r0a_v, r0b_v]
    r1 = [r1a_v, r1b_v]

    def issue(q):
        sl = pl.ds(q * QT, QT)
        return (pltpu.async_copy(od_ref.at[sk0_v.at[sl]], r0[q % 2], sem),
                pltpu.async_copy(od_ref.at[sk1_v.at[sl]], r1[q % 2], sem))

    pend = issue(0)
    for q in range(NQ):
        cur = pend
        if q + 1 < NQ:
            pend = issue(q + 1)
        cur[0].wait()
        cur[1].wait()
        UR = 4

        def row_body(rg, _, q=q):
            for u in range(UR):
                r = rg * UR + u
                a = 2 * (q * QT + r)
                w0 = plsc.load_gather(tw_v, [jnp.full((L,), a, jnp.int32)])
                w1 = plsc.load_gather(tw_v, [jnp.full((L,), a + 1, jnp.int32)])
                for jj in range(H // L):
                    sl = pl.ds(jj * L, L)
                    out_v[r, sl] = (w0 * r0[q % 2][r, sl] +
                                    w1 * r1[q % 2][r, sl])
            return 0

        lax.fori_loop(0, QT // UR, row_body, 0)
        pltpu.sync_copy(out_v, moe_ref.at[pl.ds(wid * CHT + q * QT, QT)])


def _final_body(mlp_ref, moe_ref, p1_ref, p2_ref, pf_ref, out_ref):
    def rms(v, w):
        ms = jnp.mean(v * v, axis=1, keepdims=True) + EPS
        return v * lax.rsqrt(ms) * w

    a = rms(mlp_ref[...], p1_ref[...])
    b = rms(moe_ref[...], p2_ref[...])
    out_ref[...] = rms(a + b, pf_ref[...])


def kernel(hidden_states, pre_ln_w, pre_ln2_w, post_ln1_w, post_ln2_w,
           post_ln_w, gate_w, up_w, down_w, router_w, router_scale,
           per_expert_scale, exp_gate_w, exp_up_w, exp_down_w):
    bf16 = jnp.bfloat16
    f32 = jnp.float32
    x2d = hidden_states.reshape(T, H)

    n_tt = T // TT
    h1, h2f, ti, tw, hist = pl.pallas_call(
        _stage_a_body,
        grid=(n_tt,),
        in_specs=[
            pl.BlockSpec((TT, H), lambda i: (i, 0)),
            pl.BlockSpec((1, H), lambda i: (0, 0)),
            pl.BlockSpec((1, H), lambda i: (0, 0)),
            pl.BlockSpec((H, E), lambda i: (0, 0)),
            pl.BlockSpec((1, H), lambda i: (0, 0)),
            pl.BlockSpec((1, E), lambda i: (0, 0)),
        ],
        out_specs=[
            pl.BlockSpec((TT, H), lambda i: (i, 0)),
            pl.BlockSpec((TT, H), lambda i: (i, 0)),
            pl.BlockSpec((TT, K), lambda i: (i, 0)),
            pl.BlockSpec((TT, K), lambda i: (i, 0)),
            pl.BlockSpec((1, TT // CHT, L), lambda i: (i, 0, 0)),
        ],
        out_shape=[
            jax.ShapeDtypeStruct((T, H), bf16),
            jax.ShapeDtypeStruct((T, H), f32),
            jax.ShapeDtypeStruct((T, K), jnp.int32),
            jax.ShapeDtypeStruct((T, K), f32),
            jax.ShapeDtypeStruct((T // TT, TT // CHT, L), jnp.int32),
        ],
    )(x2d, pre_ln_w.reshape(1, H), pre_ln2_w.reshape(1, H), router_w.T,
      router_scale.reshape(1, H), per_expert_scale.reshape(1, E))

    mlp_raw = pl.pallas_call(
        _glu_body,
        grid=(I // IT,),
        in_specs=[
            pl.BlockSpec((T, H), lambda i: (0, 0)),
            pl.BlockSpec((IT, H), lambda i: (i, 0)),
            pl.BlockSpec((IT, H), lambda i: (i, 0)),
            pl.BlockSpec((H, IT), lambda i: (0, i)),
        ],
        out_specs=pl.BlockSpec((T, H), lambda i: (0, 0)),
        out_shape=jax.ShapeDtypeStruct((T, H), f32),
    )(h1, gate_w, up_w, down_w)

    tif = ti.reshape(T * K)
    hist = hist.reshape(NW, L)
    twf = tw.reshape(T * K)

    xd, slots, counts = pl.kernel(
        _d2_dispatch_body,
        out_type=[
            jax.ShapeDtypeStruct((E * CAP, H), f32),
            jax.ShapeDtypeStruct((T * K,), jnp.int32),
            jax.ShapeDtypeStruct((L,), jnp.int32),
        ],
        mesh=_sc_mesh(),
        compiler_params=pltpu.CompilerParams(needs_layout_passes=False),
        scratch_types=[
            pltpu.VMEM((CHA,), jnp.int32),
            pltpu.VMEM((CHA,), jnp.int32),
            pltpu.VMEM((NW, L), jnp.int32),
            pltpu.VMEM((CHT, H), f32),
            pltpu.VMEM((CHT,), jnp.int32),
            pltpu.VMEM((CHT,), jnp.int32),
            pltpu.VMEM((L,), jnp.int32),
            pltpu.SemaphoreType.DMA,
        ],
    )(tif, h2f, hist)

    grid_spec = pltpu.PrefetchScalarGridSpec(
        num_scalar_prefetch=1,
        grid=(E, NB),
        in_specs=[
            pl.BlockSpec(
                (TILE_G, H),
                lambda e, j, c: (
                    e * NB + jnp.minimum(
                        j, jnp.maximum((c[e] + TILE_G - 1) // TILE_G - 1, 0)),
                    0)),
            pl.BlockSpec((1, MI, H), lambda e, j, c: (e, 0, 0)),
            pl.BlockSpec((1, MI, H), lambda e, j, c: (e, 0, 0)),
            pl.BlockSpec((1, H, MI), lambda e, j, c: (e, 0, 0)),
        ],
        out_specs=pl.BlockSpec(
            (TILE_G, H),
            lambda e, j, c: (
                e * NB + jnp.minimum(
                    j, jnp.maximum((c[e] + TILE_G - 1) // TILE_G - 1, 0)),
                0)),
    )
    od = pl.pallas_call(
        _grouped_body,
        grid_spec=grid_spec,
        out_shape=jax.ShapeDtypeStruct((E * CAP, H), f32),
    )(counts, xd, exp_gate_w, exp_up_w, exp_down_w)

    moe_raw = pl.kernel(
        _d3_combine_body,
        out_type=jax.ShapeDtypeStruct((T, H), f32),
        mesh=_sc_mesh(),
        compiler_params=pltpu.CompilerParams(needs_layout_passes=False),
        scratch_types=[
            pltpu.VMEM((CHA,), jnp.int32),
            pltpu.VMEM((CHA,), f32),
            pltpu.VMEM((CHT,), jnp.int32),
            pltpu.VMEM((CHT,), jnp.int32),
            pltpu.VMEM((L, H), f32),
            pltpu.VMEM((L, H), f32),
            pltpu.VMEM((L, H), f32),
            pltpu.VMEM((L, H), f32),
            pltpu.VMEM((L, H), f32),
            pltpu.SemaphoreType.DMA,
        ],
    )(slots, twf, od)

    out = pl.pallas_call(
        _final_body,
        grid=(n_tt,),
        in_specs=[
            pl.BlockSpec((TT, H), lambda i: (i, 0)),
            pl.BlockSpec((TT, H), lambda i: (i, 0)),
            pl.BlockSpec((1, H), lambda i: (0, 0)),
            pl.BlockSpec((1, H), lambda i: (0, 0)),
            pl.BlockSpec((1, H), lambda i: (0, 0)),
        ],
        out_specs=pl.BlockSpec((TT, H), lambda i: (i, 0)),
        out_shape=jax.ShapeDtypeStruct((T, H), f32),
    )(mlp_raw, moe_raw, post_ln1_w.reshape(1, H), post_ln2_w.reshape(1, H),
      post_ln_w.reshape(1, H))

    return out.reshape(B, S, H)
